# baseline (speedup 1.0000x reference)
import functools

import jax
import jax.numpy as jnp
from jax import lax
from jax.experimental import pallas as pl
from jax.experimental.pallas import tpu as pltpu
from jax.experimental.pallas import tpu_sc as plsc

D = 64
NC, NS = 2, 16
NW = NC * NS
CHUNK = 800
LANES = 16
BIG = 0x7FFFFFFF


def _embed_call(xf, table):
    B = xf.shape[0]
    bpw = B // NW
    nchunk = bpw // CHUNK
    vregs_per_chunk = CHUNK // LANES

    mesh = plsc.VectorSubcoreMesh(
        core_axis_name="c", subcore_axis_name="s", num_cores=NC, num_subcores=NS
    )

    @functools.partial(
        pl.kernel,
        out_type=jax.ShapeDtypeStruct((B, D), jnp.float32),
        mesh=mesh,
        compiler_params=pltpu.CompilerParams(use_tc_tiling_on_sc=False),
        scratch_types=[
            pltpu.VMEM((CHUNK,), jnp.int32),
            pltpu.VMEM((CHUNK,), jnp.int32),
            pltpu.VMEM((CHUNK, D), jnp.float32),
            pltpu.VMEM((CHUNK, D), jnp.float32),
            pltpu.SemaphoreType.DMA,
            pltpu.SemaphoreType.DMA,
            pltpu.SemaphoreType.DMA,
            pltpu.SemaphoreType.DMA,
            pltpu.SemaphoreType.DMA,
            pltpu.SemaphoreType.DMA,
        ],
    )
    def run(x_hbm, table_hbm, out_hbm, idxb0, idxb1, data0, data1,
            s_i0, s_i1, s_g0, s_g1, s_w0, s_w1):
        wid = lax.axis_index("s") * NC + lax.axis_index("c")
        base = wid * bpw
        idxb = (idxb0, idxb1)
        data = (data0, data1)
        s_i = (s_i0, s_i1)
        s_g = (s_g0, s_g1)
        s_w = (s_w0, s_w1)

        def start_idx(g, b):
            pltpu.async_copy(x_hbm.at[pl.ds(base + g * CHUNK, CHUNK)], idxb[b], s_i[b])

        def wait_idx(b):
            pltpu.make_async_copy(x_hbm.at[pl.ds(0, CHUNK)], idxb[b], s_i[b]).wait()

        def start_gather(b):
            pltpu.async_copy(table_hbm.at[idxb[b]], data[b], s_g[b])

        def wait_gather(b):
            pltpu.make_async_copy(table_hbm.at[idxb[b]], data[b], s_g[b]).wait()

        def start_write(g, b):
            pltpu.async_copy(
                data[b], out_hbm.at[pl.ds(base + g * CHUNK, CHUNK)], s_w[b]
            )

        def wait_write(b):
            pltpu.make_async_copy(
                data[b], out_hbm.at[pl.ds(0, CHUNK)], s_w[b]
            ).wait()

        lane = lax.iota(jnp.int32, LANES)
        zero16 = jnp.zeros((LANES,), jnp.float32)

        def fix_padding_rows(b):
            def scan(j, vmin):
                v = idxb[b][pl.ds(j * LANES, LANES)]
                return jnp.minimum(vmin, jnp.where(v == 0, j * LANES + lane, BIG))

            vmin = lax.fori_loop(
                0, vregs_per_chunk, scan, jnp.full((LANES,), BIG, jnp.int32)
            )
            fzp = vmin[0]
            for i in range(1, LANES):
                fzp = jnp.minimum(fzp, vmin[i])

            @pl.when(fzp != BIG)
            def _zero_rows():
                def body(j, carry):
                    v = idxb[b][pl.ds(j * LANES, LANES)]
                    for i in range(LANES):
                        @pl.when(v[i] == 0)
                        def _clear():
                            row = j * LANES + i
                            for k in range(D // LANES):
                                data[b][row, pl.ds(k * LANES, LANES)] = zero16

                    return carry

                lax.fori_loop(0, vregs_per_chunk, body, 0)

        start_idx(0, 0)
        start_idx(1, 1)
        wait_idx(0)
        start_gather(0)

        def outer(gg, carry):
            for b in (0, 1):
                g = gg * 2 + b
                wait_gather(b)
                fix_padding_rows(b)

                @pl.when(g >= 1)
                def _drain_prev_write():
                    wait_write(1 - b)

                start_write(g, b)

                @pl.when(g + 1 < nchunk)
                def _start_next_gather():
                    wait_idx(1 - b)
                    start_gather(1 - b)

                @pl.when(g + 2 < nchunk)
                def _start_next_idx():
                    start_idx(g + 2, b)

            return carry

        assert nchunk % 2 == 0
        lax.fori_loop(0, nchunk // 2, outer, 0)
        wait_write((nchunk - 1) % 2)

    return run(xf, table)


def kernel(x, table):
    b, l = x.shape
    xf = x.reshape(-1).astype(jnp.int32)
    half = xf.shape[0] // 2
    o1 = _embed_call(xf[:half], table)
    o2 = _embed_call(xf[half:], table)
    return jnp.concatenate([o1, o2], axis=0).reshape(b, l, D)


# speedup vs baseline: 1.1961x; 1.1961x over previous
import functools

import jax
import jax.numpy as jnp
from jax import lax
from jax.experimental import pallas as pl
from jax.experimental.pallas import tpu as pltpu
from jax.experimental.pallas import tpu_sc as plsc

D = 64
NC, NS = 2, 16
NW = NC * NS
CHUNK = 800
LANES = 16
BIG = 0x7FFFFFFF


def _embed_call(xf, table):
    B = xf.shape[0]
    bpw = B // NW
    nchunk = bpw // CHUNK
    vregs_per_chunk = CHUNK // LANES

    mesh = plsc.VectorSubcoreMesh(
        core_axis_name="c", subcore_axis_name="s", num_cores=NC, num_subcores=NS
    )

    @functools.partial(
        pl.kernel,
        out_type=jax.ShapeDtypeStruct((B, D), jnp.float32),
        mesh=mesh,
        compiler_params=pltpu.CompilerParams(use_tc_tiling_on_sc=False),
        scratch_types=[
            pltpu.VMEM((CHUNK,), jnp.int32),
            pltpu.VMEM((CHUNK,), jnp.int32),
            pltpu.VMEM((CHUNK, D), jnp.float32),
            pltpu.VMEM((CHUNK, D), jnp.float32),
            pltpu.SemaphoreType.DMA,
            pltpu.SemaphoreType.DMA,
            pltpu.SemaphoreType.DMA,
            pltpu.SemaphoreType.DMA,
            pltpu.SemaphoreType.DMA,
            pltpu.SemaphoreType.DMA,
        ],
    )
    def run(x_hbm, table_hbm, out_hbm, idxb0, idxb1, data0, data1,
            s_i0, s_i1, s_g0, s_g1, s_w0, s_w1):
        wid = lax.axis_index("s") * NC + lax.axis_index("c")
        base = wid * bpw
        idxb = (idxb0, idxb1)
        data = (data0, data1)
        s_i = (s_i0, s_i1)
        s_g = (s_g0, s_g1)
        s_w = (s_w0, s_w1)

        def start_idx(g, b):
            pltpu.async_copy(x_hbm.at[pl.ds(base + g * CHUNK, CHUNK)], idxb[b], s_i[b])

        def wait_idx(b):
            pltpu.make_async_copy(x_hbm.at[pl.ds(0, CHUNK)], idxb[b], s_i[b]).wait()

        def start_gather(b):
            pltpu.async_copy(table_hbm.at[idxb[b]], data[b], s_g[b])

        def wait_gather(b):
            pltpu.make_async_copy(table_hbm.at[idxb[b]], data[b], s_g[b]).wait()

        def start_write(g, b):
            pltpu.async_copy(
                data[b], out_hbm.at[pl.ds(base + g * CHUNK, CHUNK)], s_w[b]
            )

        def wait_write(b):
            pltpu.make_async_copy(
                data[b], out_hbm.at[pl.ds(0, CHUNK)], s_w[b]
            ).wait()

        lane = lax.iota(jnp.int32, LANES)
        zero16 = jnp.zeros((LANES,), jnp.float32)

        def fix_padding_rows(b):
            def scan(j, vmin):
                v = idxb[b][pl.ds(j * LANES, LANES)]
                return jnp.minimum(vmin, jnp.where(v == 0, j * LANES + lane, BIG))

            vmin = lax.fori_loop(
                0, vregs_per_chunk, scan, jnp.full((LANES,), BIG, jnp.int32)
            )
            fzp = vmin[0]
            for i in range(1, LANES):
                fzp = jnp.minimum(fzp, vmin[i])

            @pl.when(fzp != BIG)
            def _zero_rows():
                def body(j, carry):
                    v = idxb[b][pl.ds(j * LANES, LANES)]
                    for i in range(LANES):
                        @pl.when(v[i] == 0)
                        def _clear():
                            row = j * LANES + i
                            for k in range(D // LANES):
                                data[b][row, pl.ds(k * LANES, LANES)] = zero16

                    return carry

                lax.fori_loop(0, vregs_per_chunk, body, 0)

        start_idx(0, 0)
        start_idx(1, 1)
        wait_idx(0)
        start_gather(0)

        def outer(gg, carry):
            for b in (0, 1):
                g = gg * 2 + b
                wait_gather(b)
                fix_padding_rows(b)

                @pl.when(g >= 1)
                def _drain_prev_write():
                    wait_write(1 - b)

                start_write(g, b)

                @pl.when(g + 1 < nchunk)
                def _start_next_gather():
                    wait_idx(1 - b)
                    start_gather(1 - b)

                @pl.when(g + 2 < nchunk)
                def _start_next_idx():
                    start_idx(g + 2, b)

            return carry

        assert nchunk % 2 == 0
        lax.fori_loop(0, nchunk // 2, outer, 0)
        wait_write((nchunk - 1) % 2)

    return run(xf, table)


def kernel(x, table):
    b, l = x.shape
    xf = x.reshape(-1).astype(jnp.int32)
    out = _embed_call(xf, table)
    return out.reshape(b, l, D)
